# Initial kernel scaffold; baseline (speedup 1.0000x reference)
import jax
import jax.numpy as jnp
from jax.experimental import pallas as pl


def kernel(pe_temporal, pe_causal, pe_epistemic, pe_deontic, causal_depth, epistemic_len, deontic_len):
    raise NotImplementedError("write your pallas kernel here")


# speedup vs baseline: 1.5637x; 1.5637x over previous
import functools

import jax
import jax.numpy as jnp
from jax import lax
from jax.experimental import pallas as pl
from jax.experimental.pallas import tpu as pltpu
from jax.experimental.pallas import tpu_sc as plsc


@functools.lru_cache(maxsize=None)
def _build_sc_call(B, D4, n_t, n_c, n_e, n_d):
    info = plsc.get_sparse_core_info()
    NC, NS = info.num_cores, info.num_subcores
    NW = NC * NS
    rows_w = B // NW
    CHUNK = 64
    n_chunks = rows_w // CHUNK

    mesh = plsc.VectorSubcoreMesh(core_axis_name="c", subcore_axis_name="s")

    @functools.partial(
        pl.kernel,
        out_type=jax.ShapeDtypeStruct((B, 4 * D4), jnp.float32),
        mesh=mesh,
        scratch_types=[
            pltpu.VMEM((rows_w,), jnp.int32),
            pltpu.VMEM((rows_w,), jnp.int32),
            pltpu.VMEM((rows_w,), jnp.int32),
            pltpu.VMEM((2, CHUNK), jnp.int32),
            pltpu.VMEM((2, CHUNK), jnp.int32),
            pltpu.VMEM((2, CHUNK), jnp.int32),
            pltpu.VMEM((2, CHUNK), jnp.int32),
            pltpu.VMEM((2, CHUNK, D4), jnp.float32),
            pltpu.VMEM((2, CHUNK, D4), jnp.float32),
            pltpu.VMEM((2, CHUNK, D4), jnp.float32),
            pltpu.VMEM((2, CHUNK, D4), jnp.float32),
            pltpu.SemaphoreType.DMA,
            pltpu.SemaphoreType.DMA,
            pltpu.SemaphoreType.DMA,
        ],
    )
    def body(tbl_t, tbl_c, tbl_e, tbl_d, cdep, elen, dlen, out,
             cbuf, ebuf, dbuf, ix_t, ix_c, ix_e, ix_d,
             dst_t, dst_c, dst_e, dst_d, gsem, wsem0, wsem1):
        tables = (tbl_t, tbl_c, tbl_e, tbl_d)
        ixs = (ix_t, ix_c, ix_e, ix_d)
        dsts = (dst_t, dst_c, dst_e, dst_d)
        wsems = (wsem0, wsem1)

        wid = lax.axis_index("s") * NC + lax.axis_index("c")
        obase = wid * rows_w
        pltpu.sync_copy(cdep.at[pl.ds(obase, rows_w)], cbuf)
        pltpu.sync_copy(elen.at[pl.ds(obase, rows_w)], ebuf)
        pltpu.sync_copy(dlen.at[pl.ds(obase, rows_w)], dbuf)

        lane = lax.iota(jnp.int32, 16)
        write_handles = [None] * n_chunks
        for ci in range(n_chunks):
            p = ci % 2
            if ci >= 2:
                for h in write_handles[ci - 2]:
                    h.wait()
            g0 = obase + ci * CHUNK
            for j in range(CHUNK // 16):
                o = ci * CHUNK + j * 16
                sl = pl.ds(j * 16, 16)
                r = g0 + (j * 16) + lane
                ix_t[p, sl] = lax.rem(r, n_t)
                ix_c[p, sl] = jnp.minimum(cbuf[pl.ds(o, 16)], n_c - 1)
                ix_e[p, sl] = jnp.minimum(ebuf[pl.ds(o, 16)], n_e - 1)
                ix_d[p, sl] = jnp.minimum(dbuf[pl.ds(o, 16)], n_d - 1)
            ghs = [
                pltpu.async_copy(tb.at[ix.at[p]], db.at[p], gsem)
                for tb, ix, db in zip(tables, ixs, dsts)
            ]
            for h in ghs:
                h.wait()
            write_handles[ci] = [
                pltpu.async_copy(
                    db.at[p],
                    out.at[pl.ds(g0, CHUNK), pl.ds(s * D4, D4)],
                    wsems[p])
                for s, db in enumerate(dsts)
            ]
        for ci in (n_chunks - 2, n_chunks - 1):
            for h in write_handles[ci]:
                h.wait()

    return body


def kernel(pe_temporal, pe_causal, pe_epistemic, pe_deontic,
           causal_depth, epistemic_len, deontic_len):
    n_t, d4 = pe_temporal.shape
    n_c = pe_causal.shape[0]
    n_e = pe_epistemic.shape[0]
    n_d = pe_deontic.shape[0]
    B = causal_depth.shape[0]
    call = _build_sc_call(B, d4, n_t, n_c, n_e, n_d)
    return call(pe_temporal, pe_causal, pe_epistemic, pe_deontic,
                causal_depth.astype(jnp.int32),
                epistemic_len.astype(jnp.int32),
                deontic_len.astype(jnp.int32))
